# baseline (speedup 1.0000x reference)
import functools

import jax
import jax.numpy as jnp
from jax import lax
from jax.experimental import pallas as pl
from jax.experimental.pallas import tpu as pltpu
from jax.experimental.pallas import tpu_sc as plsc

N = 10000
E = 320000
D = 128
H = 128
R = 8

NC = 2
NS = 16
NW = NC * NS
C = 64
RPT = 160
EPAD = NW * RPT * C
NRC = 81920
CPT = NRC // NS
NPAD = 10240
APT = NPAD // NS
ZB = 1024
PK = 16384

NB = 4
LA = 2

_mesh = plsc.VectorSubcoreMesh(core_axis_name="c", subcore_axis_name="s")
_sc_params = pltpu.CompilerParams(use_tc_tiling_on_sc=False,
                                  needs_layout_passes=False)


@functools.partial(
    pl.kernel,
    out_type=(
        jax.ShapeDtypeStruct((NW, RPT, C), jnp.int32),
        jax.ShapeDtypeStruct((NW, RPT, C), jnp.float32),
    ),
    mesh=_mesh,
    compiler_params=_sc_params,
    scratch_types=(
        [pltpu.VMEM_SHARED((NRC,), jnp.float32),
         pltpu.VMEM((RPT, C), jnp.int32),
         pltpu.VMEM((RPT, C), jnp.int32),
         pltpu.VMEM((RPT, C), jnp.int32),
         pltpu.VMEM((RPT, C), jnp.float32),
         pltpu.VMEM((C,), jnp.float32),
         pltpu.VMEM((ZB,), jnp.float32)]
        + [pltpu.VMEM((C,), jnp.float32) for _ in range(4)]
        + [pltpu.SemaphoreType.DMA for _ in range(5)]
    ),
)
def _prep(src3, dst3, et3, packed_out, norm_out,
          cnt, dstb, etb, srcb, nrmb, ones, zbuf,
          cv0, cv1, cv2, cv3, cs, ns0, ns1, ns2, ns3):
    cv = (cv0, cv1, cv2, cv3)
    ns = (ns0, ns1, ns2, ns3)
    c = lax.axis_index("c")
    s = lax.axis_index("s")
    wid = c * NS + s

    def zrow(i, carry):
        zbuf[pl.ds(i * 16, 16)] = jnp.zeros((16,), jnp.float32)
        return carry

    lax.fori_loop(0, ZB // 16, zrow, 0)
    for t in range(CPT // ZB):
        pltpu.sync_copy(zbuf, cnt.at[pl.ds(s * CPT + t * ZB, ZB)])
    for v in range(C // 16):
        ones[pl.ds(v * 16, 16)] = jnp.full((16,), 1.0, jnp.float32)
    plsc.subcore_barrier()

    for half in range(2):
        pltpu.sync_copy(dst3.at[2 * s + half], dstb)
        pltpu.sync_copy(et3.at[2 * s + half], etb)

        def crow(j, carry):
            for v in range(C // 16):
                sl = pl.ds(v * 16, 16)
                etb[j, sl] = dstb[j, sl] * R + etb[j, sl]
            pltpu.async_copy(ones, cnt.at[etb.at[j]], cs, add=True)
            return carry

        lax.fori_loop(0, RPT, crow, 0)

        def cdrain(j, carry):
            pltpu.make_async_copy(ones, cnt.at[etb.at[j]], cs).wait()
            return carry

        lax.fori_loop(0, RPT, cdrain, 0)
    plsc.subcore_barrier()

    pltpu.sync_copy(src3.at[wid], srcb)
    pltpu.sync_copy(dst3.at[wid], dstb)
    pltpu.sync_copy(et3.at[wid], etb)

    def prow(j, carry):
        for v in range(C // 16):
            sl = pl.ds(v * 16, 16)
            srcb[j, sl] = (etb[j, sl] * N + srcb[j, sl]) * PK + dstb[j, sl]
            etb[j, sl] = dstb[j, sl] * R + etb[j, sl]
        return carry

    lax.fori_loop(0, RPT, prow, 0)

    for u in range(2):
        pltpu.async_copy(cnt.at[etb.at[u]], cv[u], ns[u])

    def ngrp(p, carry):
        for u in range(4):
            j = p * 4 + u
            jn = j + 2
            un = (u + 2) % 4

            @pl.when(jn < RPT)
            def _issue_next():
                pltpu.async_copy(cnt.at[etb.at[jn]], cv[un], ns[un])

            pltpu.make_async_copy(cnt.at[etb.at[j]], cv[u], ns[u]).wait()
            for v in range(C // 16):
                sl = pl.ds(v * 16, 16)
                nrmb[j, sl] = 1.0 / jnp.maximum(cv[u][sl], 1.0)
        return carry

    lax.fori_loop(0, RPT // 4, ngrp, 0)
    pltpu.sync_copy(srcb, packed_out.at[wid])
    pltpu.sync_copy(nrmb, norm_out.at[wid])


@functools.partial(
    pl.kernel,
    out_type=jax.ShapeDtypeStruct((NC, NPAD, H), jnp.float32),
    mesh=_mesh,
    compiler_params=_sc_params,
    scratch_types=(
        [pltpu.VMEM_SHARED((NPAD, H), jnp.float32),
         pltpu.VMEM((RPT, C), jnp.int32)]
        + [pltpu.VMEM((C, H), jnp.float32) for _ in range(NB)]
        + [pltpu.VMEM((C,), jnp.int32) for _ in range(NB)]
        + [pltpu.VMEM((C,), jnp.int32) for _ in range(NB)]
        + [pltpu.VMEM((C,), jnp.float32) for _ in range(NB)]
        + [pltpu.SemaphoreType.DMA for _ in range(2 * NB)]
    ),
)
def _agg(hr, packed3, norm3, parts, acc, pb,
         b0, b1, b2, b3, k0, k1, k2, k3, d0, d1, d2, d3,
         n0, n1, n2, n3, g0, g1, g2, g3, t0, t1, t2, t3):
    bufs = (b0, b1, b2, b3)
    skr = (k0, k1, k2, k3)
    dkr = (d0, d1, d2, d3)
    nr = (n0, n1, n2, n3)
    gs = (g0, g1, g2, g3)
    ss = (t0, t1, t2, t3)
    c = lax.axis_index("c")
    s = lax.axis_index("s")
    wid = c * NS + s

    def zrow(i, carry):
        for k in range(H // 16):
            b0[i, pl.ds(k * 16, 16)] = jnp.zeros((16,), jnp.float32)
        return carry

    lax.fori_loop(0, C, zrow, 0)
    for t in range(APT // C):
        pltpu.sync_copy(b0, acc.at[pl.ds(s * APT + t * C, C)])
    pltpu.sync_copy(packed3.at[wid], pb)
    plsc.subcore_barrier()

    def unpack(row, q):
        for v in range(C // 16):
            sl = pl.ds(v * 16, 16)
            w = pb[row, sl]
            skr[q][sl] = lax.shift_right_logical(w, 14)
            dkr[q][sl] = lax.bitwise_and(w, PK - 1)

    def issue(jn, q):
        unpack(jn, q)
        pltpu.async_copy(hr.at[skr[q]], bufs[q], gs[q])
        pltpu.async_copy(norm3.at[wid, jn], nr[q], gs[q])

    for q in range(LA):
        issue(q, q)

    def group(p, carry):
        for q in range(NB):
            j = p * NB + q
            jn = j + LA
            qn = (q + LA) % NB

            @pl.when(jn < RPT)
            def _issue_next():
                @pl.when(j >= NB - LA)
                def _wait_scat():
                    pltpu.make_async_copy(bufs[qn], acc.at[dkr[qn]],
                                          ss[qn]).wait()
                issue(jn, qn)

            pltpu.make_async_copy(hr.at[skr[q]], bufs[q], gs[q]).wait()
            pltpu.make_async_copy(norm3.at[wid, 0], nr[q], gs[q]).wait()

            def scale(e, carry2, _q=q):
                nv = plsc.load_gather(nr[_q],
                                      [e + jnp.zeros((16,), jnp.int32)])
                for k in range(H // 16):
                    sl = pl.ds(k * 16, 16)
                    bufs[_q][e, sl] = bufs[_q][e, sl] * nv
                return carry2

            lax.fori_loop(0, C, scale, 0, unroll=8)
            pltpu.async_copy(bufs[q], acc.at[dkr[q]], ss[q], add=True)
        return carry

    lax.fori_loop(0, RPT // NB, group, 0)
    for q in range(NB):
        pltpu.make_async_copy(bufs[q], acc.at[dkr[q]], ss[q]).wait()
    plsc.subcore_barrier()
    for t in range(APT // C):
        pltpu.sync_copy(acc.at[pl.ds(s * APT + t * C, C)], b0)
        pltpu.sync_copy(b0, parts.at[c, pl.ds(s * APT + t * C, C)])


def _mmrel_body(x_ref, w_ref, o_ref):
    o_ref[...] = jnp.dot(x_ref[...], w_ref[0],
                         preferred_element_type=jnp.float32)


def _matmul_rel(x, w_rel, bn=1000):
    n, d = x.shape
    r, _, m = w_rel.shape
    nb = n // bn
    return pl.pallas_call(
        _mmrel_body,
        grid=(nb, r),
        in_specs=[pl.BlockSpec((bn, d), lambda i, rr: (i, 0)),
                  pl.BlockSpec((1, d, m), lambda i, rr: (rr, 0, 0))],
        out_specs=pl.BlockSpec((bn, m), lambda i, rr: (rr * nb + i, 0)),
        out_shape=jax.ShapeDtypeStruct((r * n, m), jnp.float32),
    )(x, w_rel)


def _fused_body(x_ref, wroot_ref, p_ref, wrel_ref, oh_ref, ohr_ref):
    h = jnp.dot(x_ref[...], wroot_ref[...], preferred_element_type=jnp.float32)
    h = jnp.maximum(h + p_ref[0] + p_ref[1], 0.0)
    oh_ref[...] = h
    for r in range(R):
        ohr_ref[r] = jnp.dot(h, wrel_ref[r],
                             preferred_element_type=jnp.float32)


def _combine_mm(x, wroot, parts, wrel, bn=1000):
    nb = N // bn
    return pl.pallas_call(
        _fused_body,
        grid=(nb,),
        in_specs=[pl.BlockSpec((bn, D), lambda i: (i, 0)),
                  pl.BlockSpec((D, H), lambda i: (0, 0)),
                  pl.BlockSpec((NC, bn, H), lambda i: (0, i, 0)),
                  pl.BlockSpec((R, H, H), lambda i: (0, 0, 0))],
        out_specs=(pl.BlockSpec((bn, H), lambda i: (i, 0)),
                   pl.BlockSpec((R, bn, H), lambda i: (0, i, 0))),
        out_shape=(jax.ShapeDtypeStruct((N, H), jnp.float32),
                   jax.ShapeDtypeStruct((R, N, H), jnp.float32)),
    )(x, wroot, parts, wrel)


def _comb_body(mode, x_ref, w_ref, p_ref, o_ref):
    h = jnp.dot(x_ref[...], w_ref[...], preferred_element_type=jnp.float32)
    h = h + p_ref[0] + p_ref[1]
    if mode == "relu":
        h = jnp.maximum(h, 0.0)
    else:
        nrm = jnp.sqrt(jnp.sum(h * h, axis=-1, keepdims=True))
        h = h / jnp.maximum(nrm, 1e-12)
    o_ref[...] = h


def _combine(mode, x, w, parts, bn=1000):
    n, k = x.shape
    m = w.shape[1]
    return pl.pallas_call(
        functools.partial(_comb_body, mode),
        grid=(n // bn,),
        in_specs=[pl.BlockSpec((bn, k), lambda i: (i, 0)),
                  pl.BlockSpec((k, m), lambda i: (0, 0)),
                  pl.BlockSpec((NC, bn, m), lambda i: (0, i, 0))],
        out_specs=pl.BlockSpec((bn, m), lambda i: (i, 0)),
        out_shape=jax.ShapeDtypeStruct((n, m), jnp.float32),
    )(x, w, parts)


def kernel(x, edge_index, edge_type, W_root0, W_rel0, W_root1, W_rel1):
    npad = EPAD - E
    pad_lin = jnp.arange(EPAD - E, dtype=jnp.int32)
    src3 = jnp.concatenate(
        [edge_index[0], jax.lax.rem(pad_lin, jnp.int32(N))]).reshape(NW, RPT, C)
    pad_dst = N + jax.lax.rem(pad_lin, jnp.int32(NPAD - N))
    dst3 = jnp.concatenate([edge_index[1], pad_dst]).reshape(NW, RPT, C)
    et3 = jnp.concatenate(
        [edge_type, jnp.zeros((npad,), jnp.int32)]).reshape(NW, RPT, C)

    packed3, norm3 = _prep(src3, dst3, et3)

    hr = _matmul_rel(x, W_rel0)
    parts = _agg(hr, packed3, norm3)
    h, hr3 = _combine_mm(x, W_root0, parts, W_rel1)
    parts = _agg(hr3.reshape(R * N, H), packed3, norm3)
    return _combine("l2", h, W_root1, parts)


# speedup vs baseline: 1.0034x; 1.0034x over previous
import functools

import jax
import jax.numpy as jnp
from jax import lax
from jax.experimental import pallas as pl
from jax.experimental.pallas import tpu as pltpu
from jax.experimental.pallas import tpu_sc as plsc

N = 10000
E = 320000
D = 128
H = 128
R = 8

NC = 2
NS = 16
NW = NC * NS
C = 64
RPT = 160
EPAD = NW * RPT * C
NRC = 81920
CPT = NRC // NS
NPAD = 10240
APT = NPAD // NS
ZB = 1024
PK = 16384

NB = 4
LA = 2

_mesh = plsc.VectorSubcoreMesh(core_axis_name="c", subcore_axis_name="s")
_sc_params = pltpu.CompilerParams(use_tc_tiling_on_sc=False,
                                  needs_layout_passes=False)


@functools.partial(
    pl.kernel,
    out_type=(
        jax.ShapeDtypeStruct((NW, RPT, C), jnp.int32),
        jax.ShapeDtypeStruct((NW, RPT, C), jnp.float32),
    ),
    mesh=_mesh,
    compiler_params=_sc_params,
    scratch_types=(
        [pltpu.VMEM_SHARED((NRC,), jnp.float32),
         pltpu.VMEM((RPT, C), jnp.int32),
         pltpu.VMEM((RPT, C), jnp.int32),
         pltpu.VMEM((RPT, C), jnp.int32),
         pltpu.VMEM((RPT, C), jnp.float32),
         pltpu.VMEM((C,), jnp.float32),
         pltpu.VMEM((ZB,), jnp.float32)]
        + [pltpu.VMEM((C,), jnp.float32) for _ in range(4)]
        + [pltpu.SemaphoreType.DMA for _ in range(5)]
    ),
)
def _prep(src3, dst3, et3, packed_out, norm_out,
          cnt, dstb, etb, srcb, nrmb, ones, zbuf,
          cv0, cv1, cv2, cv3, cs, ns0, ns1, ns2, ns3):
    cv = (cv0, cv1, cv2, cv3)
    ns = (ns0, ns1, ns2, ns3)
    c = lax.axis_index("c")
    s = lax.axis_index("s")
    wid = c * NS + s

    def zrow(i, carry):
        zbuf[pl.ds(i * 16, 16)] = jnp.zeros((16,), jnp.float32)
        return carry

    lax.fori_loop(0, ZB // 16, zrow, 0)
    for t in range(CPT // ZB):
        pltpu.sync_copy(zbuf, cnt.at[pl.ds(s * CPT + t * ZB, ZB)])
    for v in range(C // 16):
        ones[pl.ds(v * 16, 16)] = jnp.full((16,), 1.0, jnp.float32)
    plsc.subcore_barrier()

    for half in range(2):
        pltpu.sync_copy(dst3.at[2 * s + half], dstb)
        pltpu.sync_copy(et3.at[2 * s + half], etb)

        def crow(j, carry):
            for v in range(C // 16):
                sl = pl.ds(v * 16, 16)
                etb[j, sl] = dstb[j, sl] * R + etb[j, sl]
            pltpu.async_copy(ones, cnt.at[etb.at[j]], cs, add=True)
            return carry

        lax.fori_loop(0, RPT, crow, 0)

        def cdrain(j, carry):
            pltpu.make_async_copy(ones, cnt.at[etb.at[j]], cs).wait()
            return carry

        lax.fori_loop(0, RPT, cdrain, 0)
    plsc.subcore_barrier()

    pltpu.sync_copy(src3.at[wid], srcb)
    pltpu.sync_copy(dst3.at[wid], dstb)
    pltpu.sync_copy(et3.at[wid], etb)

    def prow(j, carry):
        for v in range(C // 16):
            sl = pl.ds(v * 16, 16)
            srcb[j, sl] = (etb[j, sl] * N + srcb[j, sl]) * PK + dstb[j, sl]
            etb[j, sl] = dstb[j, sl] * R + etb[j, sl]
        return carry

    lax.fori_loop(0, RPT, prow, 0)

    for u in range(2):
        pltpu.async_copy(cnt.at[etb.at[u]], cv[u], ns[u])

    def ngrp(p, carry):
        for u in range(4):
            j = p * 4 + u
            jn = j + 2
            un = (u + 2) % 4

            @pl.when(jn < RPT)
            def _issue_next():
                pltpu.async_copy(cnt.at[etb.at[jn]], cv[un], ns[un])

            pltpu.make_async_copy(cnt.at[etb.at[j]], cv[u], ns[u]).wait()
            for v in range(C // 16):
                sl = pl.ds(v * 16, 16)
                nrmb[j, sl] = 1.0 / jnp.maximum(cv[u][sl], 1.0)
        return carry

    lax.fori_loop(0, RPT // 4, ngrp, 0)
    pltpu.sync_copy(srcb, packed_out.at[wid])
    pltpu.sync_copy(nrmb, norm_out.at[wid])


@functools.partial(
    pl.kernel,
    out_type=jax.ShapeDtypeStruct((NC, NPAD, H), jnp.float32),
    mesh=_mesh,
    compiler_params=_sc_params,
    scratch_types=(
        [pltpu.VMEM_SHARED((NPAD, H), jnp.float32),
         pltpu.VMEM((RPT, C), jnp.int32)]
        + [pltpu.VMEM((C, H), jnp.float32) for _ in range(NB)]
        + [pltpu.VMEM((C,), jnp.int32) for _ in range(NB)]
        + [pltpu.VMEM((C,), jnp.int32) for _ in range(NB)]
        + [pltpu.VMEM((C,), jnp.float32) for _ in range(NB)]
        + [pltpu.SemaphoreType.DMA for _ in range(2 * NB)]
    ),
)
def _agg(hr, packed3, norm3, parts, acc, pb,
         b0, b1, b2, b3, k0, k1, k2, k3, d0, d1, d2, d3,
         n0, n1, n2, n3, g0, g1, g2, g3, t0, t1, t2, t3):
    bufs = (b0, b1, b2, b3)
    skr = (k0, k1, k2, k3)
    dkr = (d0, d1, d2, d3)
    nr = (n0, n1, n2, n3)
    gs = (g0, g1, g2, g3)
    ss = (t0, t1, t2, t3)
    c = lax.axis_index("c")
    s = lax.axis_index("s")
    wid = c * NS + s

    def zrow(i, carry):
        for k in range(H // 16):
            b0[i, pl.ds(k * 16, 16)] = jnp.zeros((16,), jnp.float32)
        return carry

    lax.fori_loop(0, C, zrow, 0)
    for t in range(APT // C):
        pltpu.sync_copy(b0, acc.at[pl.ds(s * APT + t * C, C)])
    pltpu.sync_copy(packed3.at[wid], pb)
    plsc.subcore_barrier()

    def unpack(row, q):
        for v in range(C // 16):
            sl = pl.ds(v * 16, 16)
            w = pb[row, sl]
            skr[q][sl] = lax.shift_right_logical(w, 14)
            dkr[q][sl] = lax.bitwise_and(w, PK - 1)

    def issue(jn, q):
        unpack(jn, q)
        pltpu.async_copy(hr.at[skr[q]], bufs[q], gs[q])
        pltpu.async_copy(norm3.at[wid, jn], nr[q], gs[q])

    for q in range(LA):
        issue(q, q)

    def group(p, carry):
        for q in range(NB):
            j = p * NB + q
            jn = j + LA
            qn = (q + LA) % NB

            @pl.when(jn < RPT)
            def _issue_next():
                @pl.when(j >= NB - LA)
                def _wait_scat():
                    pltpu.make_async_copy(bufs[qn], acc.at[dkr[qn]],
                                          ss[qn]).wait()
                issue(jn, qn)

            pltpu.make_async_copy(hr.at[skr[q]], bufs[q], gs[q]).wait()
            pltpu.make_async_copy(norm3.at[wid, 0], nr[q], gs[q]).wait()

            def scale(e, carry2, _q=q):
                nv = plsc.load_gather(nr[_q],
                                      [e + jnp.zeros((16,), jnp.int32)])
                for k in range(H // 16):
                    sl = pl.ds(k * 16, 16)
                    bufs[_q][e, sl] = bufs[_q][e, sl] * nv
                return carry2

            lax.fori_loop(0, C, scale, 0, unroll=4)
            pltpu.async_copy(bufs[q], acc.at[dkr[q]], ss[q], add=True)
        return carry

    lax.fori_loop(0, RPT // NB, group, 0)
    for q in range(NB):
        pltpu.make_async_copy(bufs[q], acc.at[dkr[q]], ss[q]).wait()
    plsc.subcore_barrier()
    for t in range(APT // C):
        pltpu.sync_copy(acc.at[pl.ds(s * APT + t * C, C)], b0)
        pltpu.sync_copy(b0, parts.at[c, pl.ds(s * APT + t * C, C)])


def _mmrel_body(x_ref, w_ref, o_ref):
    o_ref[...] = jnp.dot(x_ref[...], w_ref[0],
                         preferred_element_type=jnp.float32)


def _matmul_rel(x, w_rel, bn=1000):
    n, d = x.shape
    r, _, m = w_rel.shape
    nb = n // bn
    return pl.pallas_call(
        _mmrel_body,
        grid=(nb, r),
        in_specs=[pl.BlockSpec((bn, d), lambda i, rr: (i, 0)),
                  pl.BlockSpec((1, d, m), lambda i, rr: (rr, 0, 0))],
        out_specs=pl.BlockSpec((bn, m), lambda i, rr: (rr * nb + i, 0)),
        out_shape=jax.ShapeDtypeStruct((r * n, m), jnp.float32),
    )(x, w_rel)


def _fused_body(x_ref, wroot_ref, p_ref, wrel_ref, oh_ref, ohr_ref):
    h = jnp.dot(x_ref[...], wroot_ref[...], preferred_element_type=jnp.float32)
    h = jnp.maximum(h + p_ref[0] + p_ref[1], 0.0)
    oh_ref[...] = h
    for r in range(R):
        ohr_ref[r] = jnp.dot(h, wrel_ref[r],
                             preferred_element_type=jnp.float32)


def _combine_mm(x, wroot, parts, wrel, bn=1000):
    nb = N // bn
    return pl.pallas_call(
        _fused_body,
        grid=(nb,),
        in_specs=[pl.BlockSpec((bn, D), lambda i: (i, 0)),
                  pl.BlockSpec((D, H), lambda i: (0, 0)),
                  pl.BlockSpec((NC, bn, H), lambda i: (0, i, 0)),
                  pl.BlockSpec((R, H, H), lambda i: (0, 0, 0))],
        out_specs=(pl.BlockSpec((bn, H), lambda i: (i, 0)),
                   pl.BlockSpec((R, bn, H), lambda i: (0, i, 0))),
        out_shape=(jax.ShapeDtypeStruct((N, H), jnp.float32),
                   jax.ShapeDtypeStruct((R, N, H), jnp.float32)),
    )(x, wroot, parts, wrel)


def _comb_body(mode, x_ref, w_ref, p_ref, o_ref):
    h = jnp.dot(x_ref[...], w_ref[...], preferred_element_type=jnp.float32)
    h = h + p_ref[0] + p_ref[1]
    if mode == "relu":
        h = jnp.maximum(h, 0.0)
    else:
        nrm = jnp.sqrt(jnp.sum(h * h, axis=-1, keepdims=True))
        h = h / jnp.maximum(nrm, 1e-12)
    o_ref[...] = h


def _combine(mode, x, w, parts, bn=1000):
    n, k = x.shape
    m = w.shape[1]
    return pl.pallas_call(
        functools.partial(_comb_body, mode),
        grid=(n // bn,),
        in_specs=[pl.BlockSpec((bn, k), lambda i: (i, 0)),
                  pl.BlockSpec((k, m), lambda i: (0, 0)),
                  pl.BlockSpec((NC, bn, m), lambda i: (0, i, 0))],
        out_specs=pl.BlockSpec((bn, m), lambda i: (i, 0)),
        out_shape=jax.ShapeDtypeStruct((n, m), jnp.float32),
    )(x, w, parts)


def kernel(x, edge_index, edge_type, W_root0, W_rel0, W_root1, W_rel1):
    npad = EPAD - E
    pad_lin = jnp.arange(EPAD - E, dtype=jnp.int32)
    src3 = jnp.concatenate(
        [edge_index[0], jax.lax.rem(pad_lin, jnp.int32(N))]).reshape(NW, RPT, C)
    pad_dst = N + jax.lax.rem(pad_lin, jnp.int32(NPAD - N))
    dst3 = jnp.concatenate([edge_index[1], pad_dst]).reshape(NW, RPT, C)
    et3 = jnp.concatenate(
        [edge_type, jnp.zeros((npad,), jnp.int32)]).reshape(NW, RPT, C)

    packed3, norm3 = _prep(src3, dst3, et3)

    hr = _matmul_rel(x, W_rel0)
    parts = _agg(hr, packed3, norm3)
    h, hr3 = _combine_mm(x, W_root0, parts, W_rel1)
    parts = _agg(hr3.reshape(R * N, H), packed3, norm3)
    return _combine("l2", h, W_root1, parts)
